# speedup vs baseline: 14.7949x; 14.7949x over previous
import jax
import jax.numpy as jnp
from jax.experimental import pallas as pl

_K = 2048
_SCALE = 16.0
_ROWS_PER_BLOCK = 8


def _bitcast_i32(x):
    return jax.lax.bitcast_convert_type(x, jnp.int32)


def _topk_mask_kernel(x_ref, o_ref):
    x = x_ref[...]
    b = _bitcast_i32(x)
    ikey = b ^ jnp.where(b < 0, jnp.int32(0x7FFFFFFF), jnp.int32(0))
    R = x.shape[0]

    def body(i, cur):
        bit = 30 - i
        cand = cur | (jnp.int32(1) << bit)
        cnt = jnp.sum(jnp.where(ikey >= cand, jnp.int32(1), jnp.int32(0)),
                      axis=1, keepdims=True)
        return jnp.where(cnt >= _K, cand, cur)

    cur0 = jnp.full((R, 1), jnp.iinfo(jnp.int32).min, dtype=jnp.int32)
    cnt_pos = jnp.sum(jnp.where(ikey >= 0, jnp.int32(1), jnp.int32(0)),
                      axis=1, keepdims=True)
    cur0 = jnp.where(cnt_pos >= _K, jnp.zeros_like(cur0), cur0)
    thr = jax.lax.fori_loop(0, 31, body, cur0)
    o_ref[...] = jnp.where(ikey >= thr, x * _SCALE, 0.0)


def kernel(x):
    B, N = x.shape
    grid = (B // _ROWS_PER_BLOCK,)
    return pl.pallas_call(
        _topk_mask_kernel,
        grid=grid,
        in_specs=[pl.BlockSpec((_ROWS_PER_BLOCK, N), lambda i: (i, 0))],
        out_specs=pl.BlockSpec((_ROWS_PER_BLOCK, N), lambda i: (i, 0)),
        out_shape=jax.ShapeDtypeStruct((B, N), x.dtype),
    )(x)
